# baseline (speedup 1.0000x reference)
import functools

import jax
import jax.numpy as jnp
from jax import lax
from jax.experimental import pallas as pl
from jax.experimental.pallas import tpu as pltpu
from jax.experimental.pallas import tpu_sc as plsc

N = 10000
E = 320000
IN_DIM = 128
H4 = 256
H8 = 512
OUT_DIM = 128

NC = 2
NS = 16
NW = NC * NS
NPAD = 10240
RPT = NPAD // NS

CHUNK = 32
NBUF = 8
EPT = 10240
EPAD = EPT * NW
NCHUNKS = EPT // CHUNK
EROWS = EPAD // CHUNK
NCH0 = NCHUNKS * 2 * 6 // 10
NCH1 = NCHUNKS * 2 - NCH0
NG0 = NCH0 // NBUF
NG1 = NCH1 // NBUF

_mesh = plsc.VectorSubcoreMesh(core_axis_name="c", subcore_axis_name="s")


def _zero_buf(buf, nrow):
    zero = jnp.zeros((16,), jnp.float32)
    def zrow(i, _):
        def zcol(j, _2):
            buf[i, pl.ds(j * 16, 16)] = zero
            return 0
        return lax.fori_loop(0, 8, zcol, 0)
    lax.fori_loop(0, nrow, zrow, 0)


@functools.partial(
    pl.kernel,
    out_type=jax.ShapeDtypeStruct((NC, NPAD, 128), jnp.float32),
    mesh=_mesh,
    scratch_types=[
        pltpu.VMEM((EPT // 128, 128), jnp.int32),
        pltpu.VMEM((128, 128), jnp.float32),
        pltpu.VMEM((64, 128), jnp.float32),
        pltpu.VMEM_SHARED((NPAD, 128), jnp.float32),
        [pltpu.SemaphoreType.DMA for _ in range(4)],
    ],
)
def _deg_kernel(dst_hbm, out_hbm, dsts_v, ones_v, zbuf_v, acc_sh, sems):
    c = lax.axis_index("c")
    s = lax.axis_index("s")
    wid = s * NC + c

    pltpu.sync_copy(dst_hbm.at[pl.ds(wid * (EPT // 128), EPT // 128)], dsts_v)

    one = jnp.ones((16,), jnp.float32)
    def fill(i, _):
        def fcol(j, _2):
            ones_v[i, pl.ds(j * 16, 16)] = one
            return 0
        return lax.fori_loop(0, 8, fcol, 0)
    lax.fori_loop(0, 128, fill, 0)
    _zero_buf(zbuf_v, 64)

    base = s * RPT
    def zc(k, _):
        pltpu.sync_copy(zbuf_v, acc_sh.at[pl.ds(base + k * 64, 64)])
        return 0
    lax.fori_loop(0, RPT // 64, zc, 0)
    plsc.subcore_barrier()

    def step(g, _):
        ds = [pltpu.async_copy(ones_v, acc_sh.at[dsts_v.at[g * 4 + b]],
                               sems[b], add=True) for b in range(4)]
        for b in range(4):
            ds[b].wait()
        return 0
    lax.fori_loop(0, EPT // 128 // 4, step, 0)
    plsc.subcore_barrier()

    def cpo(k, _):
        r = base + k * 128
        pltpu.sync_copy(acc_sh.at[pl.ds(r, 128)], ones_v)
        pltpu.sync_copy(ones_v, out_hbm.at[c, pl.ds(r, 128)])
        return 0
    lax.fori_loop(0, RPT // 128, cpo, 0)


def _agg_pipeline(table_hbm, src_hbm, dst_hbm, rows, idx, isems, gsems, ssems,
                  rowbase, nsup, ngroups, acc_sh):
    pltpu.async_copy(src_hbm.at[pl.ds(rowbase, NBUF)], idx[0][0], isems[0])
    pltpu.async_copy(dst_hbm.at[pl.ds(rowbase, NBUF)], idx[0][1], isems[0])

    def sup(t, _):
        for half in range(2):
            g = t * 2 + half
            src_v, dst_v = idx[half]
            nsrc_v, ndst_v = idx[1 - half]
            pltpu.make_async_copy(
                src_hbm.at[pl.ds(rowbase + g * NBUF, NBUF)], src_v,
                isems[half]).wait()
            pltpu.make_async_copy(
                dst_hbm.at[pl.ds(rowbase + g * NBUF, NBUF)], dst_v,
                isems[half]).wait()
            @pl.when(g + 1 < ngroups)
            def _():
                pltpu.async_copy(
                    src_hbm.at[pl.ds(rowbase + (g + 1) * NBUF, NBUF)],
                    nsrc_v, isems[1 - half])
                pltpu.async_copy(
                    dst_hbm.at[pl.ds(rowbase + (g + 1) * NBUF, NBUF)],
                    ndst_v, isems[1 - half])
            for b in range(NBUF):
                @pl.when(g > 0)
                def _():
                    pltpu.make_async_copy(
                        rows[b], acc_sh.at[ndst_v.at[b]], ssems[b]).wait()
                pltpu.async_copy(table_hbm.at[src_v.at[b]], rows[b], gsems[b])
            for b in range(NBUF):
                pltpu.make_async_copy(
                    table_hbm.at[src_v.at[b]], rows[b], gsems[b]).wait()
                pltpu.async_copy(rows[b], acc_sh.at[dst_v.at[b]],
                                 ssems[b], add=True)
        return 0
    lax.fori_loop(0, nsup, sup, 0)
    for b in range(NBUF):
        pltpu.make_async_copy(rows[b], acc_sh.at[idx[1][1].at[b]],
                              ssems[b]).wait()


_AGG_SCRATCH = [
    [pltpu.VMEM((CHUNK, 128), jnp.float32) for _ in range(NBUF)],
    [[pltpu.VMEM((NBUF, CHUNK), jnp.int32) for _ in range(2)]
     for _ in range(2)],
    pltpu.VMEM_SHARED((NPAD, 128), jnp.float32),
    [pltpu.SemaphoreType.DMA for _ in range(2)],
    [pltpu.SemaphoreType.DMA for _ in range(NBUF)],
    [pltpu.SemaphoreType.DMA for _ in range(NBUF)],
]


@functools.partial(
    pl.kernel,
    out_type=jax.ShapeDtypeStruct((NC, NPAD, 128), jnp.float32),
    mesh=_mesh,
    scratch_types=_AGG_SCRATCH,
)
def _agg_kernel(table_hbm, src_hbm, dst_hbm, out_hbm,
                rows, idx, acc_sh, isems, gsems, ssems):
    c = lax.axis_index("c")
    s = lax.axis_index("s")
    wid = s * NC + c

    _zero_buf(rows[0], 64)
    base = s * RPT
    def zc(k, _):
        pltpu.sync_copy(rows[0].at[pl.ds(0, 64)],
                        acc_sh.at[pl.ds(base + k * 64, 64)])
        return 0
    lax.fori_loop(0, RPT // 64, zc, 0)
    plsc.subcore_barrier()

    rowbase = jnp.where(c == 0, s * NCH0, NS * NCH0 + s * NCH1)
    nsup = jnp.where(c == 0, NG0 // 2, NG1 // 2)
    ngroups = jnp.where(c == 0, NG0, NG1)
    _agg_pipeline(table_hbm, src_hbm, dst_hbm, rows, idx,
                  isems, gsems, ssems, rowbase, nsup, ngroups, acc_sh)
    plsc.subcore_barrier()

    def cpo(k, _):
        r = base + k * CHUNK
        pltpu.sync_copy(acc_sh.at[pl.ds(r, CHUNK)], rows[0])
        pltpu.sync_copy(rows[0], out_hbm.at[c, pl.ds(r, CHUNK)])
        return 0
    lax.fori_loop(0, RPT // CHUNK, cpo, 0)


@functools.partial(
    pl.kernel,
    out_type=jax.ShapeDtypeStruct((2, NC, NPAD, 128), jnp.float32),
    mesh=_mesh,
    scratch_types=_AGG_SCRATCH,
)
def _agg2_kernel(lo_hbm, hi_hbm, src_hbm, dst_hbm, out_hbm,
                 rows, idx, acc_sh, isems, gsems, ssems):
    c = lax.axis_index("c")
    s = lax.axis_index("s")
    wid = s * NC + c
    base = s * RPT

    def zc(k, _):
        pltpu.sync_copy(rows[0].at[pl.ds(0, 64)],
                        acc_sh.at[pl.ds(base + k * 64, 64)])
        return 0

    def cpo(h):
        def body(k, _):
            r = base + k * CHUNK
            pltpu.sync_copy(acc_sh.at[pl.ds(r, CHUNK)], rows[0])
            pltpu.sync_copy(rows[0], out_hbm.at[h, c, pl.ds(r, CHUNK)])
            return 0
        return body

    for h, table in ((0, lo_hbm), (1, hi_hbm)):
        _zero_buf(rows[0], 64)
        lax.fori_loop(0, RPT // 64, zc, 0)
        plsc.subcore_barrier()
        _agg_pipeline(table, src_hbm, dst_hbm, rows, idx,
                      isems, gsems, ssems,
                      jnp.where(c == 0, s * NCH0, NS * NCH0 + s * NCH1),
                      jnp.where(c == 0, NG0 // 2, NG1 // 2),
                      jnp.where(c == 0, NG0, NG1), acc_sh)
        plsc.subcore_barrier()
        lax.fori_loop(0, RPT // CHUNK, cpo(h), 0)
        plsc.subcore_barrier()


_B = 1000
_G = N // _B

def _ln(y, g, b, eps=1e-5):
    mu = jnp.mean(y, axis=-1, keepdims=True)
    var = jnp.mean((y - mu) ** 2, axis=-1, keepdims=True)
    return (y - mu) / jnp.sqrt(var + eps) * g + b


def _tc1_body(degp_ref, x_ref, wfc_ref, bfc_ref, g_ref, b_ref,
              xs_ref, f1_ref, dis_ref):
    deg = degp_ref[0, :, 0:1] + degp_ref[1, :, 0:1]
    dis = lax.rsqrt(deg + 1.0)
    x = x_ref[...]
    xs_ref[...] = x * dis
    dis_ref[...] = jnp.broadcast_to(dis, (_B, 128))
    y = jnp.dot(x, wfc_ref[...], preferred_element_type=jnp.float32) + bfc_ref[...]
    f1_ref[...] = jnp.maximum(_ln(y, g_ref[...], b_ref[...]), 0.0)


def _tc2_body(z1p_ref, xs_ref, dis_ref, w1_ref, w2_ref, f1_ref,
              lo_ref, hi_ref):
    z1 = z1p_ref[0] + z1p_ref[1] + xs_ref[...]
    a1 = dis_ref[...] * z1
    x1 = jnp.maximum(jnp.dot(a1, w1_ref[...], preferred_element_type=jnp.float32), 0.0)
    h2 = (jnp.dot(x1, w2_ref[0:H4], preferred_element_type=jnp.float32)
          + jnp.dot(f1_ref[...], w2_ref[H4:H8], preferred_element_type=jnp.float32))
    h2s = h2 * dis_ref[:, :1]
    lo_ref[...] = h2s[:, 0:128]
    hi_ref[...] = h2s[:, 128:256]


def _tc3_body(z2p_ref, hlo_ref, hhi_ref, dis_ref,
              b2_ref, g2_ref, b2n_ref, w3_ref, h3s_ref):
    zlo = z2p_ref[0, 0] + z2p_ref[0, 1] + hlo_ref[...]
    zhi = z2p_ref[1, 0] + z2p_ref[1, 1] + hhi_ref[...]
    z2 = jnp.concatenate([zlo, zhi], axis=1)
    u2 = z2 * dis_ref[:, :1] + b2_ref[...]
    x2 = jnp.maximum(_ln(u2, g2_ref[...], b2n_ref[...]), 0.0)
    h3 = jnp.dot(x2, w3_ref[...], preferred_element_type=jnp.float32)
    h3s_ref[...] = h3 * dis_ref[...]


def _tc4_body(z3p_ref, h3s_ref, dis_ref, b3_ref, g3_ref, b3n_ref, out_ref):
    z3 = z3p_ref[0] + z3p_ref[1] + h3s_ref[...]
    u3 = dis_ref[...] * z3 + b3_ref[...]
    out_ref[...] = _ln(u3, g3_ref[...], b3n_ref[...])


def _rows_spec(w):
    return pl.BlockSpec((_B, w), lambda i: (i, 0))

def _parts(w):
    return pl.BlockSpec((NC, _B, w), lambda i: (0, i, 0))

def _full(a, b):
    return pl.BlockSpec((a, b), lambda i: (0, 0))


_tc1 = pl.pallas_call(
    _tc1_body,
    grid=(_G,),
    in_specs=[_parts(128), _rows_spec(128), _full(IN_DIM, H4),
              _full(1, H4), _full(1, H4), _full(1, H4)],
    out_specs=[_rows_spec(128), _rows_spec(H4), _rows_spec(128)],
    out_shape=[jax.ShapeDtypeStruct((N, 128), jnp.float32),
               jax.ShapeDtypeStruct((N, H4), jnp.float32),
               jax.ShapeDtypeStruct((N, 128), jnp.float32)],
)

_tc2 = pl.pallas_call(
    _tc2_body,
    grid=(_G,),
    in_specs=[_parts(128), _rows_spec(128), _rows_spec(128),
              _full(128, H4), _full(H8, H4), _rows_spec(H4)],
    out_specs=[_rows_spec(128), _rows_spec(128)],
    out_shape=[jax.ShapeDtypeStruct((N, 128), jnp.float32),
               jax.ShapeDtypeStruct((N, 128), jnp.float32)],
)

_tc3 = pl.pallas_call(
    _tc3_body,
    grid=(_G,),
    in_specs=[pl.BlockSpec((2, NC, _B, 128), lambda i: (0, 0, i, 0)),
              _rows_spec(128), _rows_spec(128), _rows_spec(128),
              _full(1, H4), _full(1, H4), _full(1, H4), _full(H4, OUT_DIM)],
    out_specs=[_rows_spec(128)],
    out_shape=[jax.ShapeDtypeStruct((N, 128), jnp.float32)],
)

_tc4 = pl.pallas_call(
    _tc4_body,
    grid=(_G,),
    in_specs=[_parts(128), _rows_spec(128), _rows_spec(128),
              _full(1, OUT_DIM), _full(1, OUT_DIM), _full(1, OUT_DIM)],
    out_specs=[_rows_spec(OUT_DIM)],
    out_shape=[jax.ShapeDtypeStruct((N, OUT_DIM), jnp.float32)],
)


def kernel(x, edge_index, W1, Wfc, bfc, gfc_g, gfc_b, W2, b2, g2, b2n, W3, b3, g3, b3n):
    src = edge_index[0]
    dst = edge_index[1]
    pad = EPAD - E
    srcp = jnp.concatenate([src, jnp.zeros((pad,), src.dtype)]).reshape(-1, CHUNK)
    dstp = jnp.concatenate([dst, jnp.full((pad,), N, dst.dtype)]).reshape(-1, CHUNK)
    dstp128 = dstp.reshape(-1, 128)

    degp = _deg_kernel(dstp128)
    xs, f1, dis = _tc1(degp, x, Wfc,
                       bfc.reshape(1, -1), gfc_g.reshape(1, -1),
                       gfc_b.reshape(1, -1))
    z1p = _agg_kernel(xs, srcp, dstp)
    h2lo, h2hi = _tc2(z1p, xs, dis, W1, W2, f1)
    z2p = _agg2_kernel(h2lo, h2hi, srcp, dstp)
    h3s, = _tc3(z2p, h2lo, h2hi, dis,
                b2.reshape(1, -1), g2.reshape(1, -1), b2n.reshape(1, -1), W3)
    z3p = _agg_kernel(h3s, srcp, dstp)
    out, = _tc4(z3p, h3s, dis,
                b3.reshape(1, -1), g3.reshape(1, -1), b3n.reshape(1, -1))
    return out


# speedup vs baseline: 1.0216x; 1.0216x over previous
import functools

import jax
import jax.numpy as jnp
from jax import lax
from jax.experimental import pallas as pl
from jax.experimental.pallas import tpu as pltpu
from jax.experimental.pallas import tpu_sc as plsc

N = 10000
E = 320000
IN_DIM = 128
H4 = 256
H8 = 512
OUT_DIM = 128

NC = 2
NS = 16
NW = NC * NS
NPAD = 10240
RPT = NPAD // NS

CHUNK = 32
NBUF = 8
EPT = 10240
EPAD = EPT * NW
NCHUNKS = EPT // CHUNK
EROWS = EPAD // CHUNK
NCH0 = NCHUNKS * 2 * 7 // 10
NCH1 = NCHUNKS * 2 - NCH0
NG0 = NCH0 // NBUF
NG1 = NCH1 // NBUF

_mesh = plsc.VectorSubcoreMesh(core_axis_name="c", subcore_axis_name="s")


def _zero_buf(buf, nrow):
    zero = jnp.zeros((16,), jnp.float32)
    def zrow(i, _):
        def zcol(j, _2):
            buf[i, pl.ds(j * 16, 16)] = zero
            return 0
        return lax.fori_loop(0, 8, zcol, 0)
    lax.fori_loop(0, nrow, zrow, 0)


@functools.partial(
    pl.kernel,
    out_type=jax.ShapeDtypeStruct((NC, NPAD, 128), jnp.float32),
    mesh=_mesh,
    scratch_types=[
        pltpu.VMEM((EPT // 128, 128), jnp.int32),
        pltpu.VMEM((128, 128), jnp.float32),
        pltpu.VMEM((64, 128), jnp.float32),
        pltpu.VMEM_SHARED((NPAD, 128), jnp.float32),
        [pltpu.SemaphoreType.DMA for _ in range(4)],
    ],
)
def _deg_kernel(dst_hbm, out_hbm, dsts_v, ones_v, zbuf_v, acc_sh, sems):
    c = lax.axis_index("c")
    s = lax.axis_index("s")
    wid = s * NC + c

    pltpu.sync_copy(dst_hbm.at[pl.ds(wid * (EPT // 128), EPT // 128)], dsts_v)

    one = jnp.ones((16,), jnp.float32)
    def fill(i, _):
        def fcol(j, _2):
            ones_v[i, pl.ds(j * 16, 16)] = one
            return 0
        return lax.fori_loop(0, 8, fcol, 0)
    lax.fori_loop(0, 128, fill, 0)
    _zero_buf(zbuf_v, 64)

    base = s * RPT
    def zc(k, _):
        pltpu.sync_copy(zbuf_v, acc_sh.at[pl.ds(base + k * 64, 64)])
        return 0
    lax.fori_loop(0, RPT // 64, zc, 0)
    plsc.subcore_barrier()

    def step(g, _):
        ds = [pltpu.async_copy(ones_v, acc_sh.at[dsts_v.at[g * 4 + b]],
                               sems[b], add=True) for b in range(4)]
        for b in range(4):
            ds[b].wait()
        return 0
    lax.fori_loop(0, EPT // 128 // 4, step, 0)
    plsc.subcore_barrier()

    def cpo(k, _):
        r = base + k * 128
        pltpu.sync_copy(acc_sh.at[pl.ds(r, 128)], ones_v)
        pltpu.sync_copy(ones_v, out_hbm.at[c, pl.ds(r, 128)])
        return 0
    lax.fori_loop(0, RPT // 128, cpo, 0)


def _agg_pipeline(table_hbm, src_hbm, dst_hbm, rows, idx, isems, gsems, ssems,
                  rowbase, nsup, ngroups, acc_sh):
    pltpu.async_copy(src_hbm.at[pl.ds(rowbase, NBUF)], idx[0][0], isems[0])
    pltpu.async_copy(dst_hbm.at[pl.ds(rowbase, NBUF)], idx[0][1], isems[0])

    def sup(t, _):
        for half in range(2):
            g = t * 2 + half
            src_v, dst_v = idx[half]
            nsrc_v, ndst_v = idx[1 - half]
            pltpu.make_async_copy(
                src_hbm.at[pl.ds(rowbase + g * NBUF, NBUF)], src_v,
                isems[half]).wait()
            pltpu.make_async_copy(
                dst_hbm.at[pl.ds(rowbase + g * NBUF, NBUF)], dst_v,
                isems[half]).wait()
            @pl.when(g + 1 < ngroups)
            def _():
                pltpu.async_copy(
                    src_hbm.at[pl.ds(rowbase + (g + 1) * NBUF, NBUF)],
                    nsrc_v, isems[1 - half])
                pltpu.async_copy(
                    dst_hbm.at[pl.ds(rowbase + (g + 1) * NBUF, NBUF)],
                    ndst_v, isems[1 - half])
            for b in range(NBUF):
                @pl.when(g > 0)
                def _():
                    pltpu.make_async_copy(
                        rows[b], acc_sh.at[ndst_v.at[b]], ssems[b]).wait()
                pltpu.async_copy(table_hbm.at[src_v.at[b]], rows[b], gsems[b])
            for b in range(NBUF):
                pltpu.make_async_copy(
                    table_hbm.at[src_v.at[b]], rows[b], gsems[b]).wait()
                pltpu.async_copy(rows[b], acc_sh.at[dst_v.at[b]],
                                 ssems[b], add=True)
        return 0
    lax.fori_loop(0, nsup, sup, 0)
    for b in range(NBUF):
        pltpu.make_async_copy(rows[b], acc_sh.at[idx[1][1].at[b]],
                              ssems[b]).wait()


_AGG_SCRATCH = [
    [pltpu.VMEM((CHUNK, 128), jnp.float32) for _ in range(NBUF)],
    [[pltpu.VMEM((NBUF, CHUNK), jnp.int32) for _ in range(2)]
     for _ in range(2)],
    pltpu.VMEM_SHARED((NPAD, 128), jnp.float32),
    [pltpu.SemaphoreType.DMA for _ in range(2)],
    [pltpu.SemaphoreType.DMA for _ in range(NBUF)],
    [pltpu.SemaphoreType.DMA for _ in range(NBUF)],
]


@functools.partial(
    pl.kernel,
    out_type=jax.ShapeDtypeStruct((NC, NPAD, 128), jnp.float32),
    mesh=_mesh,
    scratch_types=_AGG_SCRATCH,
)
def _agg_kernel(table_hbm, src_hbm, dst_hbm, out_hbm,
                rows, idx, acc_sh, isems, gsems, ssems):
    c = lax.axis_index("c")
    s = lax.axis_index("s")
    wid = s * NC + c

    _zero_buf(rows[0], 64)
    base = s * RPT
    def zc(k, _):
        pltpu.sync_copy(rows[0].at[pl.ds(0, 64)],
                        acc_sh.at[pl.ds(base + k * 64, 64)])
        return 0
    lax.fori_loop(0, RPT // 64, zc, 0)
    plsc.subcore_barrier()

    rowbase = jnp.where(c == 0, s * NCH0, NS * NCH0 + s * NCH1)
    nsup = jnp.where(c == 0, NG0 // 2, NG1 // 2)
    ngroups = jnp.where(c == 0, NG0, NG1)
    _agg_pipeline(table_hbm, src_hbm, dst_hbm, rows, idx,
                  isems, gsems, ssems, rowbase, nsup, ngroups, acc_sh)
    plsc.subcore_barrier()

    def cpo(k, _):
        r = base + k * CHUNK
        pltpu.sync_copy(acc_sh.at[pl.ds(r, CHUNK)], rows[0])
        pltpu.sync_copy(rows[0], out_hbm.at[c, pl.ds(r, CHUNK)])
        return 0
    lax.fori_loop(0, RPT // CHUNK, cpo, 0)


@functools.partial(
    pl.kernel,
    out_type=jax.ShapeDtypeStruct((2, NC, NPAD, 128), jnp.float32),
    mesh=_mesh,
    scratch_types=_AGG_SCRATCH,
)
def _agg2_kernel(lo_hbm, hi_hbm, src_hbm, dst_hbm, out_hbm,
                 rows, idx, acc_sh, isems, gsems, ssems):
    c = lax.axis_index("c")
    s = lax.axis_index("s")
    wid = s * NC + c
    base = s * RPT

    def zc(k, _):
        pltpu.sync_copy(rows[0].at[pl.ds(0, 64)],
                        acc_sh.at[pl.ds(base + k * 64, 64)])
        return 0

    def cpo(h):
        def body(k, _):
            r = base + k * CHUNK
            pltpu.sync_copy(acc_sh.at[pl.ds(r, CHUNK)], rows[0])
            pltpu.sync_copy(rows[0], out_hbm.at[h, c, pl.ds(r, CHUNK)])
            return 0
        return body

    for h, table in ((0, lo_hbm), (1, hi_hbm)):
        _zero_buf(rows[0], 64)
        lax.fori_loop(0, RPT // 64, zc, 0)
        plsc.subcore_barrier()
        _agg_pipeline(table, src_hbm, dst_hbm, rows, idx,
                      isems, gsems, ssems,
                      jnp.where(c == 0, s * NCH0, NS * NCH0 + s * NCH1),
                      jnp.where(c == 0, NG0 // 2, NG1 // 2),
                      jnp.where(c == 0, NG0, NG1), acc_sh)
        plsc.subcore_barrier()
        lax.fori_loop(0, RPT // CHUNK, cpo(h), 0)
        plsc.subcore_barrier()


_B = 1000
_G = N // _B

def _ln(y, g, b, eps=1e-5):
    mu = jnp.mean(y, axis=-1, keepdims=True)
    var = jnp.mean((y - mu) ** 2, axis=-1, keepdims=True)
    return (y - mu) / jnp.sqrt(var + eps) * g + b


def _tc0_body(x_ref, wfc_ref, bfc_ref, g_ref, b_ref, f1_ref):
    x = x_ref[...]
    y = jnp.dot(x, wfc_ref[...], preferred_element_type=jnp.float32) + bfc_ref[...]
    f1_ref[...] = jnp.maximum(_ln(y, g_ref[...], b_ref[...]), 0.0)


def _tc1_body(degp_ref, x_ref, xs_ref, dis_ref):
    deg = degp_ref[0, :, 0:1] + degp_ref[1, :, 0:1]
    dis = lax.rsqrt(deg + 1.0)
    xs_ref[...] = x_ref[...] * dis
    dis_ref[...] = jnp.broadcast_to(dis, (_B, 128))


def _tc2_body(z1p_ref, xs_ref, dis_ref, w1_ref, w2_ref, f1_ref,
              lo_ref, hi_ref):
    z1 = z1p_ref[0] + z1p_ref[1] + xs_ref[...]
    a1 = dis_ref[...] * z1
    x1 = jnp.maximum(jnp.dot(a1, w1_ref[...], preferred_element_type=jnp.float32), 0.0)
    h2 = (jnp.dot(x1, w2_ref[0:H4], preferred_element_type=jnp.float32)
          + jnp.dot(f1_ref[...], w2_ref[H4:H8], preferred_element_type=jnp.float32))
    h2s = h2 * dis_ref[:, :1]
    lo_ref[...] = h2s[:, 0:128]
    hi_ref[...] = h2s[:, 128:256]


def _tc3_body(z2p_ref, hlo_ref, hhi_ref, dis_ref,
              b2_ref, g2_ref, b2n_ref, w3_ref, h3s_ref):
    zlo = z2p_ref[0, 0] + z2p_ref[0, 1] + hlo_ref[...]
    zhi = z2p_ref[1, 0] + z2p_ref[1, 1] + hhi_ref[...]
    z2 = jnp.concatenate([zlo, zhi], axis=1)
    u2 = z2 * dis_ref[:, :1] + b2_ref[...]
    x2 = jnp.maximum(_ln(u2, g2_ref[...], b2n_ref[...]), 0.0)
    h3 = jnp.dot(x2, w3_ref[...], preferred_element_type=jnp.float32)
    h3s_ref[...] = h3 * dis_ref[...]


def _tc4_body(z3p_ref, h3s_ref, dis_ref, b3_ref, g3_ref, b3n_ref, out_ref):
    z3 = z3p_ref[0] + z3p_ref[1] + h3s_ref[...]
    u3 = dis_ref[...] * z3 + b3_ref[...]
    out_ref[...] = _ln(u3, g3_ref[...], b3n_ref[...])


def _rows_spec(w):
    return pl.BlockSpec((_B, w), lambda i: (i, 0))

def _parts(w):
    return pl.BlockSpec((NC, _B, w), lambda i: (0, i, 0))

def _full(a, b):
    return pl.BlockSpec((a, b), lambda i: (0, 0))


_tc0 = pl.pallas_call(
    _tc0_body,
    grid=(_G,),
    in_specs=[_rows_spec(128), _full(IN_DIM, H4),
              _full(1, H4), _full(1, H4), _full(1, H4)],
    out_specs=[_rows_spec(H4)],
    out_shape=[jax.ShapeDtypeStruct((N, H4), jnp.float32)],
)

_tc1 = pl.pallas_call(
    _tc1_body,
    grid=(_G,),
    in_specs=[_parts(128), _rows_spec(128)],
    out_specs=[_rows_spec(128), _rows_spec(128)],
    out_shape=[jax.ShapeDtypeStruct((N, 128), jnp.float32),
               jax.ShapeDtypeStruct((N, 128), jnp.float32)],
)

_tc2 = pl.pallas_call(
    _tc2_body,
    grid=(_G,),
    in_specs=[_parts(128), _rows_spec(128), _rows_spec(128),
              _full(128, H4), _full(H8, H4), _rows_spec(H4)],
    out_specs=[_rows_spec(128), _rows_spec(128)],
    out_shape=[jax.ShapeDtypeStruct((N, 128), jnp.float32),
               jax.ShapeDtypeStruct((N, 128), jnp.float32)],
)

_tc3 = pl.pallas_call(
    _tc3_body,
    grid=(_G,),
    in_specs=[pl.BlockSpec((2, NC, _B, 128), lambda i: (0, 0, i, 0)),
              _rows_spec(128), _rows_spec(128), _rows_spec(128),
              _full(1, H4), _full(1, H4), _full(1, H4), _full(H4, OUT_DIM)],
    out_specs=[_rows_spec(128)],
    out_shape=[jax.ShapeDtypeStruct((N, 128), jnp.float32)],
)

_tc4 = pl.pallas_call(
    _tc4_body,
    grid=(_G,),
    in_specs=[_parts(128), _rows_spec(128), _rows_spec(128),
              _full(1, OUT_DIM), _full(1, OUT_DIM), _full(1, OUT_DIM)],
    out_specs=[_rows_spec(OUT_DIM)],
    out_shape=[jax.ShapeDtypeStruct((N, OUT_DIM), jnp.float32)],
)


def kernel(x, edge_index, W1, Wfc, bfc, gfc_g, gfc_b, W2, b2, g2, b2n, W3, b3, g3, b3n):
    src = edge_index[0]
    dst = edge_index[1]
    pad = EPAD - E
    srcp = jnp.concatenate([src, jnp.zeros((pad,), src.dtype)]).reshape(-1, CHUNK)
    dstp = jnp.concatenate([dst, jnp.full((pad,), N, dst.dtype)]).reshape(-1, CHUNK)
    dstp128 = dstp.reshape(-1, 128)

    degp = _deg_kernel(dstp128)
    f1, = _tc0(x, Wfc, bfc.reshape(1, -1), gfc_g.reshape(1, -1),
               gfc_b.reshape(1, -1))
    xs, dis = _tc1(degp, x)
    z1p = _agg_kernel(xs, srcp, dstp)
    h2lo, h2hi = _tc2(z1p, xs, dis, W1, W2, f1)
    z2p = _agg2_kernel(h2lo, h2hi, srcp, dstp)
    h3s, = _tc3(z2p, h2lo, h2hi, dis,
                b2.reshape(1, -1), g2.reshape(1, -1), b2n.reshape(1, -1), W3)
    z3p = _agg_kernel(h3s, srcp, dstp)
    out, = _tc4(z3p, h3s, dis,
                b3.reshape(1, -1), g3.reshape(1, -1), b3n.reshape(1, -1))
    return out
